# Initial kernel scaffold; baseline (speedup 1.0000x reference)
import jax
import jax.numpy as jnp
from jax.experimental import pallas as pl


def kernel(x, edge_index, W1, b1, W2, b2, W3, b3):
    raise NotImplementedError("write your pallas kernel here")


# speedup vs baseline: 12.4236x; 12.4236x over previous
import functools

import jax
import jax.numpy as jnp
from jax import lax
from jax.experimental import pallas as pl
from jax.experimental.pallas import tpu as pltpu
from jax.experimental.pallas import tpu_sc as plsc

N = 10000
E = 160000
D_IN = 256
H0 = 256
H1 = 64
D_OUT = 40

NC = 2
NS = 16
NW = NC * NS
NP = 10240
C = 128
CH = 40
EP = NW * CH * C
EPW = CH * C
RPS = NP // NS

_mesh = plsc.VectorSubcoreMesh(core_axis_name="c", subcore_axis_name="s")
_sc_params = pltpu.CompilerParams(needs_layout_passes=False)
_sc_params_lin = pltpu.CompilerParams(
    needs_layout_passes=False, use_tc_tiling_on_sc=False)


@functools.partial(
    pl.kernel,
    out_type=jax.ShapeDtypeStruct((NW, NP), jnp.float32),
    mesh=_mesh,
    compiler_params=_sc_params,
    scratch_types=[
        pltpu.VMEM((CH, C), jnp.int32),
        pltpu.VMEM((NP,), jnp.float32),
    ],
)
def _deg_kernel(dst_hbm, out_hbm, d_all, deg_v):
    cid = lax.axis_index("c")
    sid = lax.axis_index("s")
    wid = cid * NS + sid
    pltpu.sync_copy(dst_hbm.at[wid], d_all)

    def _zero(j, _):
        deg_v[pl.ds(j * 16, 16)] = jnp.zeros((16,), jnp.float32)
        return 0

    lax.fori_loop(0, NP // 16, _zero, 0)

    ones = jnp.ones((16,), jnp.float32)

    def _count(j, _):
        k = j // (C // 16)
        i = j % (C // 16)
        idx = d_all[k, pl.ds(i * 16, 16)]
        plsc.addupdate_scatter(deg_v, [idx], ones)
        return 0

    lax.fori_loop(0, CH * (C // 16), _count, 0)
    pltpu.sync_copy(deg_v, out_hbm.at[wid])


def _make_edge_kernel(nblk, F):
    def body(src_hbm, dst_hbm, *refs):
        g_hbms = refs[:nblk]
        out_hbm = refs[nblk]
        s_all, d_all, rows_v, acc_sh, sem = refs[nblk + 1:]
        cid = lax.axis_index("c")
        sid = lax.axis_index("s")
        wid = cid * NS + sid
        pltpu.sync_copy(src_hbm.at[wid], s_all)
        pltpu.sync_copy(dst_hbm.at[wid], d_all)
        for b in range(nblk):
            g_hbm = g_hbms[b]
            for j in range(RPS // C):
                pltpu.sync_copy(
                    g_hbm.at[pl.ds(N, C)],
                    acc_sh.at[pl.ds(sid * RPS + j * C, C)],
                )
            plsc.subcore_barrier()

            def _chunk(k, _):
                pltpu.async_copy(g_hbm.at[s_all.at[k]], rows_v, sem).wait()
                pltpu.sync_copy(rows_v, acc_sh.at[d_all.at[k]], add=True)
                return 0

            lax.fori_loop(0, CH, _chunk, 0)
            plsc.subcore_barrier()
            pltpu.sync_copy(
                acc_sh.at[pl.ds(sid * RPS, RPS)],
                out_hbm.at[cid, b, pl.ds(sid * RPS, RPS)],
            )
            plsc.subcore_barrier()

    return pl.kernel(
        body,
        out_type=jax.ShapeDtypeStruct((NC, nblk, NP, F), jnp.float32),
        mesh=_mesh,
        compiler_params=_sc_params if F % 128 == 0 else _sc_params_lin,
        scratch_types=[
            pltpu.VMEM((CH, C), jnp.int32),
            pltpu.VMEM((CH, C), jnp.int32),
            pltpu.VMEM((C, F), jnp.float32),
            pltpu.VMEM_SHARED((NP, F), jnp.float32),
            pltpu.SemaphoreType.DMA,
        ],
    )


_edge1 = _make_edge_kernel(2, 128)
_edge64 = _make_edge_kernel(1, 64)


def _dis_body(deg_ref, out_ref):
    tot = jnp.sum(deg_ref[...], axis=0, keepdims=True) + 1.0
    col = lax.broadcasted_iota(jnp.int32, (1, NP), 1)
    out_ref[...] = jnp.where(col < N, lax.rsqrt(tot), 0.0)


def _dis_call(degp):
    return pl.pallas_call(
        _dis_body,
        out_shape=jax.ShapeDtypeStruct((1, NP), jnp.float32),
    )(degp)


RB = 512
NRB = NP // RB


def _mm1_body(x_ref, w_ref, dis_ref, out_ref):
    hw = jnp.dot(x_ref[...], w_ref[...], preferred_element_type=jnp.float32)
    out_ref[0] = hw * dis_ref[...]


def _mm1_call(xp, W1, dis_col):
    return pl.pallas_call(
        _mm1_body,
        grid=(NRB, 2),
        in_specs=[
            pl.BlockSpec((RB, D_IN), lambda i, b: (i, 0)),
            pl.BlockSpec((D_IN, 128), lambda i, b: (0, b)),
            pl.BlockSpec((RB, 1), lambda i, b: (i, 0)),
        ],
        out_specs=pl.BlockSpec((1, RB, 128), lambda i, b: (b, i, 0)),
        out_shape=jax.ShapeDtypeStruct((2, NP, 128), jnp.float32),
    )(xp, W1, dis_col)


def _mid1_body(p_ref, g_ref, dis_ref, b_ref, w_ref, out_ref):
    dis = dis_ref[...]
    h0 = jnp.maximum(dis * (p_ref[0, 0] + p_ref[1, 0] + g_ref[0]) + b_ref[0, :128], 0.0)
    h1 = jnp.maximum(dis * (p_ref[0, 1] + p_ref[1, 1] + g_ref[1]) + b_ref[0, 128:], 0.0)
    hw = (jnp.dot(h0, w_ref[:128], preferred_element_type=jnp.float32)
          + jnp.dot(h1, w_ref[128:], preferred_element_type=jnp.float32))
    out_ref[...] = hw * dis


def _mid1_call(p1, g1, dis_col, b1r, W2):
    return pl.pallas_call(
        _mid1_body,
        grid=(NRB,),
        in_specs=[
            pl.BlockSpec((NC, 2, RB, 128), lambda i: (0, 0, i, 0)),
            pl.BlockSpec((2, RB, 128), lambda i: (0, i, 0)),
            pl.BlockSpec((RB, 1), lambda i: (i, 0)),
            pl.BlockSpec((1, H0), lambda i: (0, 0)),
            pl.BlockSpec((H0, H1), lambda i: (0, 0)),
        ],
        out_specs=pl.BlockSpec((RB, H1), lambda i: (i, 0)),
        out_shape=jax.ShapeDtypeStruct((NP, H1), jnp.float32),
    )(p1, g1, dis_col, b1r, W2)


def _mid2_body(p_ref, g_ref, dis_ref, b_ref, w_ref, out_ref):
    dis = dis_ref[...]
    h = jnp.maximum(dis * (p_ref[0, 0] + p_ref[1, 0] + g_ref[...]) + b_ref[...], 0.0)
    out_ref[...] = jnp.dot(h, w_ref[...], preferred_element_type=jnp.float32) * dis


def _mid2_call(p2, g2, dis_col, b2r, W3p):
    return pl.pallas_call(
        _mid2_body,
        grid=(NRB,),
        in_specs=[
            pl.BlockSpec((NC, 1, RB, H1), lambda i: (0, 0, i, 0)),
            pl.BlockSpec((RB, H1), lambda i: (i, 0)),
            pl.BlockSpec((RB, 1), lambda i: (i, 0)),
            pl.BlockSpec((1, H1), lambda i: (0, 0)),
            pl.BlockSpec((H1, H1), lambda i: (0, 0)),
        ],
        out_specs=pl.BlockSpec((RB, H1), lambda i: (i, 0)),
        out_shape=jax.ShapeDtypeStruct((NP, H1), jnp.float32),
    )(p2, g2, dis_col, b2r, W3p)


FRB = 400
NFRB = N // FRB


def _final_body(p_ref, g_ref, dis_ref, b_ref, out_ref):
    z = dis_ref[...] * (p_ref[0, 0] + p_ref[1, 0] + g_ref[...]) + b_ref[...]
    col = lax.broadcasted_iota(jnp.int32, (FRB, H1), 1)
    valid = col < D_OUT
    zm = jnp.where(valid, z, -jnp.inf)
    m = jnp.max(zm, axis=1, keepdims=True)
    e = jnp.where(valid, jnp.exp(z - m), 0.0)
    s = jnp.sum(e, axis=1, keepdims=True)
    out_ref[...] = (z - m - jnp.log(s))[:, :D_OUT]


def _final_call(p3, g3, dis_col, b3r):
    return pl.pallas_call(
        _final_body,
        grid=(NFRB,),
        in_specs=[
            pl.BlockSpec((NC, 1, FRB, H1), lambda i: (0, 0, i, 0)),
            pl.BlockSpec((FRB, H1), lambda i: (i, 0)),
            pl.BlockSpec((FRB, 1), lambda i: (i, 0)),
            pl.BlockSpec((1, H1), lambda i: (0, 0)),
        ],
        out_specs=pl.BlockSpec((FRB, D_OUT), lambda i: (i, 0)),
        out_shape=jax.ShapeDtypeStruct((N, D_OUT), jnp.float32),
    )(p3, g3, dis_col, b3r)


def kernel(x, edge_index, W1, b1, W2, b2, W3, b3):
    src = edge_index[0]
    dst = edge_index[1]
    padi = (N + jnp.arange(EP - E, dtype=jnp.int32) % (NP - N))
    srcp = jnp.concatenate([src, padi]).reshape(NW, CH, C)
    dstp = jnp.concatenate([dst, padi]).reshape(NW, CH, C)
    xp = jnp.pad(x, ((0, NP - N), (0, 0)))
    b1r = b1.reshape(1, H0)
    b2r = b2.reshape(1, H1)
    W3p = jnp.pad(W3, ((0, 0), (0, H1 - D_OUT)))
    b3r = jnp.pad(b3, (0, H1 - D_OUT)).reshape(1, H1)

    degp = _deg_kernel(dstp)
    dis_col = _dis_call(degp).reshape(NP, 1)

    g1 = _mm1_call(xp, W1, dis_col)
    p1 = _edge1(srcp, dstp, g1[0], g1[1])
    g2 = _mid1_call(p1, g1, dis_col, b1r, W2)
    p2 = _edge64(srcp, dstp, g2)
    g3 = _mid2_call(p2, g2, dis_col, b2r, W3p)
    p3 = _edge64(srcp, dstp, g3)
    return _final_call(p3, g3, dis_col, b3r)


# speedup vs baseline: 16.6768x; 1.3424x over previous
import functools

import jax
import jax.numpy as jnp
from jax import lax
from jax.experimental import pallas as pl
from jax.experimental.pallas import tpu as pltpu
from jax.experimental.pallas import tpu_sc as plsc

N = 10000
E = 160000
D_IN = 256
H0 = 256
H1 = 64
D_OUT = 40

NC = 2
NS = 16
NW = NC * NS
NP = 10240
C = 128
CH = 40
CHB = 80
EP = NW * CH * C
RPS = NP // NS
NB = 4

_mesh = plsc.VectorSubcoreMesh(core_axis_name="c", subcore_axis_name="s")
_sc_params = pltpu.CompilerParams(needs_layout_passes=False)
_sc_params_lin = pltpu.CompilerParams(
    needs_layout_passes=False, use_tc_tiling_on_sc=False)


@functools.partial(
    pl.kernel,
    out_type=jax.ShapeDtypeStruct((NW, NP), jnp.float32),
    mesh=_mesh,
    compiler_params=_sc_params,
    scratch_types=[
        pltpu.VMEM((CH, C), jnp.int32),
        pltpu.VMEM((NP,), jnp.float32),
    ],
)
def _deg_kernel(dst_hbm, out_hbm, d_all, deg_v):
    cid = lax.axis_index("c")
    sid = lax.axis_index("s")
    wid = cid * NS + sid
    pltpu.sync_copy(dst_hbm.at[wid], d_all)

    def _zero(j, _):
        for u in range(8):
            deg_v[pl.ds((j * 8 + u) * 16, 16)] = jnp.zeros((16,), jnp.float32)
        return 0

    lax.fori_loop(0, NP // 128, _zero, 0)

    ones = jnp.ones((16,), jnp.float32)

    def _count(k, _):
        for u in range(C // 16):
            idx = d_all[k, pl.ds(u * 16, 16)]
            plsc.addupdate_scatter(deg_v, [idx], ones)
        return 0

    lax.fori_loop(0, CH, _count, 0)
    pltpu.sync_copy(deg_v, out_hbm.at[wid])


def _ring_loop(nb, nchunks, g_hbm, s_all, d_all, rows, sems, acc_sh):
    for b in range(nb):
        pltpu.async_copy(g_hbm.at[s_all.at[b]], rows[b], sems[b])

    def _outer(k0, _):
        for b in range(nb):
            k = k0 * nb + b
            pltpu.make_async_copy(
                g_hbm.at[s_all.at[0]], rows[b], sems[b]).wait()
            pltpu.sync_copy(rows[b], acc_sh.at[d_all.at[k]], add=True)
            kn = k + nb

            @pl.when(kn < nchunks)
            def _():
                pltpu.async_copy(g_hbm.at[s_all.at[kn]], rows[b], sems[b])
        return 0

    lax.fori_loop(0, nchunks // nb, _outer, 0)


def _zero_acc_slice(g_hbm, acc_sh, sid):
    for j in range(RPS // C):
        pltpu.sync_copy(
            g_hbm.at[pl.ds(N, C)],
            acc_sh.at[pl.ds(sid * RPS + j * C, C)],
        )


NB1 = 2


def _edge1_body(srcb_hbm, dstb_hbm, g_hbm, out_hbm, s_all, d_all, *rest):
    rows = rest[:NB1]
    acc_sh = rest[NB1]
    sems = rest[NB1 + 1:]
    cid = lax.axis_index("c")
    sid = lax.axis_index("s")
    _zero_acc_slice(g_hbm, acc_sh, sid)
    plsc.subcore_barrier()
    for half in range(2):
        pltpu.sync_copy(srcb_hbm.at[cid, sid, half], s_all)
        pltpu.sync_copy(dstb_hbm.at[sid, half], d_all)
        _ring_loop(NB1, CH, g_hbm, s_all, d_all, rows, sems, acc_sh)
    plsc.subcore_barrier()
    pltpu.sync_copy(
        acc_sh.at[pl.ds(sid * RPS, RPS)],
        out_hbm.at[cid, pl.ds(sid * RPS, RPS)],
    )


_edge1 = pl.kernel(
    _edge1_body,
    out_type=jax.ShapeDtypeStruct((NC, NP, 128), jnp.float32),
    mesh=_mesh,
    compiler_params=_sc_params,
    scratch_types=[
        pltpu.VMEM((CH, C), jnp.int32),
        pltpu.VMEM((CH, C), jnp.int32),
    ] + [pltpu.VMEM((C, 128), jnp.float32) for _ in range(NB1)] + [
        pltpu.VMEM_SHARED((NP, 128), jnp.float32),
    ] + [pltpu.SemaphoreType.DMA for _ in range(NB1)],
)


def _edge64_body(src_hbm, dst_hbm, g_hbm, out_hbm, s_all, d_all, *rest):
    rows = rest[:NB]
    acc_sh = rest[NB]
    sems = rest[NB + 1:]
    cid = lax.axis_index("c")
    sid = lax.axis_index("s")
    wid = cid * NS + sid
    pltpu.sync_copy(src_hbm.at[wid], s_all)
    pltpu.sync_copy(dst_hbm.at[wid], d_all)
    _zero_acc_slice(g_hbm, acc_sh, sid)
    plsc.subcore_barrier()
    _ring_loop(NB, CH, g_hbm, s_all, d_all, rows, sems, acc_sh)
    plsc.subcore_barrier()
    pltpu.sync_copy(
        acc_sh.at[pl.ds(sid * RPS, RPS)],
        out_hbm.at[cid, pl.ds(sid * RPS, RPS)],
    )


_edge64 = pl.kernel(
    _edge64_body,
    out_type=jax.ShapeDtypeStruct((NC, NP, H1), jnp.float32),
    mesh=_mesh,
    compiler_params=_sc_params_lin,
    scratch_types=[
        pltpu.VMEM((CH, C), jnp.int32),
        pltpu.VMEM((CH, C), jnp.int32),
    ] + [pltpu.VMEM((C, H1), jnp.float32) for _ in range(NB)] + [
        pltpu.VMEM_SHARED((NP, H1), jnp.float32),
    ] + [pltpu.SemaphoreType.DMA for _ in range(NB)],
)


def _dis_body(deg_ref, out_ref):
    tot = jnp.sum(deg_ref[...], axis=0, keepdims=True) + 1.0
    col = lax.broadcasted_iota(jnp.int32, (1, NP), 1)
    out_ref[...] = jnp.where(col < N, lax.rsqrt(tot), 0.0)


def _dis_call(degp):
    return pl.pallas_call(
        _dis_body,
        out_shape=jax.ShapeDtypeStruct((1, NP), jnp.float32),
    )(degp)


RB = 512
NRB = NP // RB


def _mm1_body(x_ref, w_ref, out_ref):
    out_ref[0] = jnp.dot(x_ref[...], w_ref[...],
                         preferred_element_type=jnp.float32)


def _mm1_call(xp, W1):
    return pl.pallas_call(
        _mm1_body,
        grid=(NRB, 2),
        in_specs=[
            pl.BlockSpec((RB, D_IN), lambda i, b: (i, 0)),
            pl.BlockSpec((D_IN, 128), lambda i, b: (0, b)),
        ],
        out_specs=pl.BlockSpec((1, RB, 128), lambda i, b: (b, i, 0)),
        out_shape=jax.ShapeDtypeStruct((2, NP, 128), jnp.float32),
    )(xp, W1)


def _scale_body(hw_ref, dis_ref, out_ref):
    out_ref[0] = hw_ref[0] * dis_ref[...]


def _scale_call(hw, dis_col):
    return pl.pallas_call(
        _scale_body,
        grid=(NRB, 2),
        in_specs=[
            pl.BlockSpec((1, RB, 128), lambda i, b: (b, i, 0)),
            pl.BlockSpec((RB, 1), lambda i, b: (i, 0)),
        ],
        out_specs=pl.BlockSpec((1, RB, 128), lambda i, b: (b, i, 0)),
        out_shape=jax.ShapeDtypeStruct((2, NP, 128), jnp.float32),
    )(hw, dis_col)


def _mid1_body(p_ref, g_ref, dis_ref, b_ref, w_ref, out_ref):
    dis = dis_ref[...]
    h0 = jnp.maximum(dis * (p_ref[0] + g_ref[0]) + b_ref[0, :128], 0.0)
    h1 = jnp.maximum(dis * (p_ref[1] + g_ref[1]) + b_ref[0, 128:], 0.0)
    hw = (jnp.dot(h0, w_ref[:128], preferred_element_type=jnp.float32)
          + jnp.dot(h1, w_ref[128:], preferred_element_type=jnp.float32))
    out_ref[...] = hw * dis


def _mid1_call(p1, g1, dis_col, b1r, W2):
    return pl.pallas_call(
        _mid1_body,
        grid=(NRB,),
        in_specs=[
            pl.BlockSpec((NC, RB, 128), lambda i: (0, i, 0)),
            pl.BlockSpec((2, RB, 128), lambda i: (0, i, 0)),
            pl.BlockSpec((RB, 1), lambda i: (i, 0)),
            pl.BlockSpec((1, H0), lambda i: (0, 0)),
            pl.BlockSpec((H0, H1), lambda i: (0, 0)),
        ],
        out_specs=pl.BlockSpec((RB, H1), lambda i: (i, 0)),
        out_shape=jax.ShapeDtypeStruct((NP, H1), jnp.float32),
    )(p1, g1, dis_col, b1r, W2)


def _mid2_body(p_ref, g_ref, dis_ref, b_ref, w_ref, out_ref):
    dis = dis_ref[...]
    h = jnp.maximum(dis * (p_ref[0] + p_ref[1] + g_ref[...]) + b_ref[...], 0.0)
    out_ref[...] = jnp.dot(h, w_ref[...], preferred_element_type=jnp.float32) * dis


def _mid2_call(p2, g2, dis_col, b2r, W3p):
    return pl.pallas_call(
        _mid2_body,
        grid=(NRB,),
        in_specs=[
            pl.BlockSpec((NC, RB, H1), lambda i: (0, i, 0)),
            pl.BlockSpec((RB, H1), lambda i: (i, 0)),
            pl.BlockSpec((RB, 1), lambda i: (i, 0)),
            pl.BlockSpec((1, H1), lambda i: (0, 0)),
            pl.BlockSpec((H1, H1), lambda i: (0, 0)),
        ],
        out_specs=pl.BlockSpec((RB, H1), lambda i: (i, 0)),
        out_shape=jax.ShapeDtypeStruct((NP, H1), jnp.float32),
    )(p2, g2, dis_col, b2r, W3p)


FRB = 400
NFRB = N // FRB


def _final_body(p_ref, g_ref, dis_ref, b_ref, out_ref):
    z = dis_ref[...] * (p_ref[0] + p_ref[1] + g_ref[...]) + b_ref[...]
    col = lax.broadcasted_iota(jnp.int32, (FRB, H1), 1)
    valid = col < D_OUT
    zm = jnp.where(valid, z, -jnp.inf)
    m = jnp.max(zm, axis=1, keepdims=True)
    e = jnp.where(valid, jnp.exp(z - m), 0.0)
    s = jnp.sum(e, axis=1, keepdims=True)
    out_ref[...] = (z - m - jnp.log(s))[:, :D_OUT]


def _final_call(p3, g3, dis_col, b3r):
    return pl.pallas_call(
        _final_body,
        grid=(NFRB,),
        in_specs=[
            pl.BlockSpec((NC, FRB, H1), lambda i: (0, i, 0)),
            pl.BlockSpec((FRB, H1), lambda i: (i, 0)),
            pl.BlockSpec((FRB, 1), lambda i: (i, 0)),
            pl.BlockSpec((1, H1), lambda i: (0, 0)),
        ],
        out_specs=pl.BlockSpec((FRB, D_OUT), lambda i: (i, 0)),
        out_shape=jax.ShapeDtypeStruct((N, D_OUT), jnp.float32),
    )(p3, g3, dis_col, b3r)


def kernel(x, edge_index, W1, b1, W2, b2, W3, b3):
    src = edge_index[0]
    dst = edge_index[1]
    padi = (N + jnp.arange(EP - E, dtype=jnp.int32) % (NP - N))
    srcp = jnp.concatenate([src, padi])
    dstp = jnp.concatenate([dst, padi])
    src16 = srcp.reshape(NS, 2, CH, C)
    srcb = jnp.stack([src16, src16 + NP])
    dstb = dstp.reshape(NS, 2, CH, C)
    src32 = srcp.reshape(NW, CH, C)
    dst32 = dstp.reshape(NW, CH, C)
    xp = jnp.pad(x, ((0, NP - N), (0, 0)))
    b1r = b1.reshape(1, H0)
    b2r = b2.reshape(1, H1)
    W3p = jnp.pad(W3, ((0, 0), (0, H1 - D_OUT)))
    b3r = jnp.pad(b3, (0, H1 - D_OUT)).reshape(1, H1)

    hw1 = _mm1_call(xp, W1)
    degp = _deg_kernel(dst32)
    dis_col = _dis_call(degp).reshape(NP, 1)

    g1 = _scale_call(hw1, dis_col)
    p1 = _edge1(srcb, dstb, g1.reshape(NC * NP, 128))
    g2 = _mid1_call(p1, g1, dis_col, b1r, W2)
    p2 = _edge64(src32, dst32, g2)
    g3 = _mid2_call(p2, g2, dis_col, b2r, W3p)
    p3 = _edge64(src32, dst32, g3)
    return _final_call(p3, g3, dis_col, b3r)


# speedup vs baseline: 19.4678x; 1.1674x over previous
import functools

import jax
import jax.numpy as jnp
from jax import lax
from jax.experimental import pallas as pl
from jax.experimental.pallas import tpu as pltpu
from jax.experimental.pallas import tpu_sc as plsc

N = 10000
E = 160000
D_IN = 256
H0 = 256
H1 = 64
D_OUT = 40

NC = 2
NS = 16
NW = NC * NS
NP = 10240
C = 128
CH = 40
C1 = 64
CH1 = 40
PH1 = 4
EP = NW * CH * C
RPS = NP // NS

_mesh = plsc.VectorSubcoreMesh(core_axis_name="c", subcore_axis_name="s")
_sc_params = pltpu.CompilerParams(needs_layout_passes=False)
_sc_params_lin = pltpu.CompilerParams(
    needs_layout_passes=False, use_tc_tiling_on_sc=False)


@functools.partial(
    pl.kernel,
    out_type=jax.ShapeDtypeStruct((NW, NP), jnp.float32),
    mesh=_mesh,
    compiler_params=_sc_params,
    scratch_types=[
        pltpu.VMEM((CH, C), jnp.int32),
        pltpu.VMEM((NP,), jnp.float32),
    ],
)
def _deg_kernel(dst_hbm, out_hbm, d_all, deg_v):
    cid = lax.axis_index("c")
    sid = lax.axis_index("s")
    wid = cid * NS + sid
    pltpu.sync_copy(dst_hbm.at[wid], d_all)

    def _zero(j, _):
        for u in range(8):
            deg_v[pl.ds((j * 8 + u) * 16, 16)] = jnp.zeros((16,), jnp.float32)
        return 0

    lax.fori_loop(0, NP // 128, _zero, 0)

    ones = jnp.ones((16,), jnp.float32)

    def _count(k, _):
        for u in range(C // 16):
            idx = d_all[k, pl.ds(u * 16, 16)]
            plsc.addupdate_scatter(deg_v, [idx], ones)
        return 0

    lax.fori_loop(0, CH, _count, 0)
    pltpu.sync_copy(deg_v, out_hbm.at[wid])


def _ring_loop(nb, lag, nchunks, g_hbm, s_all, d_all, rows, gsems, ssems,
               acc_sh):
    for b in range(nb):
        pltpu.async_copy(g_hbm.at[s_all.at[b]], rows[b], gsems[b])

    def _outer(k0, _):
        for b in range(nb):
            k = k0 * nb + b
            pltpu.make_async_copy(
                g_hbm.at[s_all.at[0]], rows[b], gsems[b]).wait()
            pltpu.async_copy(rows[b], acc_sh.at[d_all.at[k]], ssems[b],
                             add=True)

            @pl.when(k >= lag)
            def _():
                bl = (b - lag) % nb
                pltpu.make_async_copy(
                    rows[bl], acc_sh.at[d_all.at[0]], ssems[bl]).wait()
                kn = k - lag + nb

                @pl.when(kn < nchunks)
                def _():
                    pltpu.async_copy(g_hbm.at[s_all.at[kn]], rows[bl],
                                     gsems[bl])
        return 0

    lax.fori_loop(0, nchunks // nb, _outer, 0)
    for j in range(lag):
        b = (nchunks - lag + j) % nb
        pltpu.make_async_copy(
            rows[b], acc_sh.at[d_all.at[0]], ssems[b]).wait()


def _zero_acc_slice(z_hbm, acc_sh, sid):
    for j in range(RPS // C):
        pltpu.sync_copy(z_hbm, acc_sh.at[pl.ds(sid * RPS + j * C, C)])


NB1 = 4


def _edge1_body(srcb_hbm, dstb_hbm, g_hbm, z_hbm, out_hbm, s_all, d_all,
                *rest):
    rows = rest[:NB1]
    acc_sh = rest[NB1]
    gsems = rest[NB1 + 1:2 * NB1 + 1]
    ssems = rest[2 * NB1 + 1:]
    cid = lax.axis_index("c")
    sid = lax.axis_index("s")
    _zero_acc_slice(z_hbm, acc_sh, sid)
    plsc.subcore_barrier()
    for half in range(PH1):
        pltpu.sync_copy(srcb_hbm.at[cid, sid, half], s_all)
        pltpu.sync_copy(dstb_hbm.at[sid, half], d_all)
        _ring_loop(NB1, 1, CH1, g_hbm, s_all, d_all, rows, gsems, ssems,
                   acc_sh)
    plsc.subcore_barrier()
    pltpu.sync_copy(
        acc_sh.at[pl.ds(sid * RPS, RPS)],
        out_hbm.at[cid, pl.ds(sid * RPS, RPS)],
    )


_edge1 = pl.kernel(
    _edge1_body,
    out_type=jax.ShapeDtypeStruct((NC, NP, 128), jnp.float32),
    mesh=_mesh,
    compiler_params=_sc_params,
    scratch_types=[
        pltpu.VMEM((CH1, C1), jnp.int32),
        pltpu.VMEM((CH1, C1), jnp.int32),
    ] + [pltpu.VMEM((C1, 128), jnp.float32) for _ in range(NB1)] + [
        pltpu.VMEM_SHARED((NP, 128), jnp.float32),
    ] + [pltpu.SemaphoreType.DMA for _ in range(2 * NB1)],
)


NB2 = 5


def _edge64_body(src_hbm, dst_hbm, g_hbm, z_hbm, out_hbm, s_all, d_all,
                 *rest):
    rows = rest[:NB2]
    acc_sh = rest[NB2]
    gsems = rest[NB2 + 1:2 * NB2 + 1]
    ssems = rest[2 * NB2 + 1:]
    cid = lax.axis_index("c")
    sid = lax.axis_index("s")
    wid = cid * NS + sid
    pltpu.sync_copy(src_hbm.at[wid], s_all)
    pltpu.sync_copy(dst_hbm.at[wid], d_all)
    _zero_acc_slice(z_hbm, acc_sh, sid)
    plsc.subcore_barrier()
    _ring_loop(NB2, 2, CH, g_hbm, s_all, d_all, rows, gsems, ssems, acc_sh)
    plsc.subcore_barrier()
    pltpu.sync_copy(
        acc_sh.at[pl.ds(sid * RPS, RPS)],
        out_hbm.at[cid, pl.ds(sid * RPS, RPS)],
    )


_edge64 = pl.kernel(
    _edge64_body,
    out_type=jax.ShapeDtypeStruct((NC, NP, H1), jnp.float32),
    mesh=_mesh,
    compiler_params=_sc_params_lin,
    scratch_types=[
        pltpu.VMEM((CH, C), jnp.int32),
        pltpu.VMEM((CH, C), jnp.int32),
    ] + [pltpu.VMEM((C, H1), jnp.float32) for _ in range(NB2)] + [
        pltpu.VMEM_SHARED((NP, H1), jnp.float32),
    ] + [pltpu.SemaphoreType.DMA for _ in range(2 * NB2)],
)


def _dis_body(deg_ref, out_ref):
    tot = jnp.sum(deg_ref[...], axis=0, keepdims=True) + 1.0
    col = lax.broadcasted_iota(jnp.int32, (1, NP), 1)
    out_ref[...] = jnp.where(col < N, lax.rsqrt(tot), 0.0)


def _dis_call(degp):
    return pl.pallas_call(
        _dis_body,
        out_shape=jax.ShapeDtypeStruct((1, NP), jnp.float32),
    )(degp)


RB = 2000
NRB = N // RB


def _mm1_body(x_ref, w_ref, out_ref):
    out_ref[0] = jnp.dot(x_ref[...], w_ref[...],
                         preferred_element_type=jnp.float32)


def _mm1_call(x, W1):
    return pl.pallas_call(
        _mm1_body,
        grid=(NRB, 2),
        in_specs=[
            pl.BlockSpec((RB, D_IN), lambda i, b: (i, 0)),
            pl.BlockSpec((D_IN, 128), lambda i, b: (0, b)),
        ],
        out_specs=pl.BlockSpec((1, RB, 128), lambda i, b: (b, i, 0)),
        out_shape=jax.ShapeDtypeStruct((2, NP, 128), jnp.float32),
    )(x, W1)


def _scale_body(hw_ref, dis_ref, out_ref):
    out_ref[0] = hw_ref[0] * dis_ref[...]


def _scale_call(hw, dis_col):
    return pl.pallas_call(
        _scale_body,
        grid=(NRB, 2),
        in_specs=[
            pl.BlockSpec((1, RB, 128), lambda i, b: (b, i, 0)),
            pl.BlockSpec((RB, 1), lambda i, b: (i, 0)),
        ],
        out_specs=pl.BlockSpec((1, RB, 128), lambda i, b: (b, i, 0)),
        out_shape=jax.ShapeDtypeStruct((2, NP, 128), jnp.float32),
    )(hw, dis_col)


def _mid1_body(p_ref, g_ref, dis_ref, b_ref, w_ref, out_ref):
    dis = dis_ref[...]
    h0 = jnp.maximum(dis * (p_ref[0] + g_ref[0]) + b_ref[0, :128], 0.0)
    h1 = jnp.maximum(dis * (p_ref[1] + g_ref[1]) + b_ref[0, 128:], 0.0)
    hw = (jnp.dot(h0, w_ref[:128], preferred_element_type=jnp.float32)
          + jnp.dot(h1, w_ref[128:], preferred_element_type=jnp.float32))
    out_ref[...] = hw * dis


def _mid1_call(p1, g1, dis_col, b1r, W2):
    return pl.pallas_call(
        _mid1_body,
        grid=(NRB,),
        in_specs=[
            pl.BlockSpec((NC, RB, 128), lambda i: (0, i, 0)),
            pl.BlockSpec((2, RB, 128), lambda i: (0, i, 0)),
            pl.BlockSpec((RB, 1), lambda i: (i, 0)),
            pl.BlockSpec((1, H0), lambda i: (0, 0)),
            pl.BlockSpec((H0, H1), lambda i: (0, 0)),
        ],
        out_specs=pl.BlockSpec((RB, H1), lambda i: (i, 0)),
        out_shape=jax.ShapeDtypeStruct((NP, H1), jnp.float32),
    )(p1, g1, dis_col, b1r, W2)


def _mid2_body(p_ref, g_ref, dis_ref, b_ref, w_ref, out_ref):
    dis = dis_ref[...]
    h = jnp.maximum(dis * (p_ref[0] + p_ref[1] + g_ref[...]) + b_ref[...], 0.0)
    out_ref[...] = jnp.dot(h, w_ref[...], preferred_element_type=jnp.float32) * dis


def _mid2_call(p2, g2, dis_col, b2r, W3p):
    return pl.pallas_call(
        _mid2_body,
        grid=(NRB,),
        in_specs=[
            pl.BlockSpec((NC, RB, H1), lambda i: (0, i, 0)),
            pl.BlockSpec((RB, H1), lambda i: (i, 0)),
            pl.BlockSpec((RB, 1), lambda i: (i, 0)),
            pl.BlockSpec((1, H1), lambda i: (0, 0)),
            pl.BlockSpec((H1, H1), lambda i: (0, 0)),
        ],
        out_specs=pl.BlockSpec((RB, H1), lambda i: (i, 0)),
        out_shape=jax.ShapeDtypeStruct((NP, H1), jnp.float32),
    )(p2, g2, dis_col, b2r, W3p)


def _final_body(p_ref, g_ref, dis_ref, b_ref, out_ref):
    z = dis_ref[...] * (p_ref[0] + p_ref[1] + g_ref[...]) + b_ref[...]
    col = lax.broadcasted_iota(jnp.int32, (RB, H1), 1)
    valid = col < D_OUT
    zm = jnp.where(valid, z, -jnp.inf)
    m = jnp.max(zm, axis=1, keepdims=True)
    e = jnp.where(valid, jnp.exp(z - m), 0.0)
    s = jnp.sum(e, axis=1, keepdims=True)
    out_ref[...] = (z - m - jnp.log(s))[:, :D_OUT]


def _final_call(p3, g3, dis_col, b3r):
    return pl.pallas_call(
        _final_body,
        grid=(NRB,),
        in_specs=[
            pl.BlockSpec((NC, RB, H1), lambda i: (0, i, 0)),
            pl.BlockSpec((RB, H1), lambda i: (i, 0)),
            pl.BlockSpec((RB, 1), lambda i: (i, 0)),
            pl.BlockSpec((1, H1), lambda i: (0, 0)),
        ],
        out_specs=pl.BlockSpec((RB, D_OUT), lambda i: (i, 0)),
        out_shape=jax.ShapeDtypeStruct((N, D_OUT), jnp.float32),
    )(p3, g3, dis_col, b3r)


def kernel(x, edge_index, W1, b1, W2, b2, W3, b3):
    src = edge_index[0]
    dst = edge_index[1]
    padi = (N + jnp.arange(EP - E, dtype=jnp.int32) % (NP - N))
    srcp = jnp.concatenate([src, padi])
    dstp = jnp.concatenate([dst, padi])
    src16 = srcp.reshape(NS, PH1, CH1, C1)
    srcb = jnp.stack([src16, src16 + NP])
    dstb = dstp.reshape(NS, PH1, CH1, C1)
    src32 = srcp.reshape(NW, CH, C)
    dst32 = dstp.reshape(NW, CH, C)
    z128 = jnp.zeros((C, 128), jnp.float32)
    z64 = jnp.zeros((C, H1), jnp.float32)
    b1r = b1.reshape(1, H0)
    b2r = b2.reshape(1, H1)
    W3p = jnp.pad(W3, ((0, 0), (0, H1 - D_OUT)))
    b3r = jnp.pad(b3, (0, H1 - D_OUT)).reshape(1, H1)

    hw1 = _mm1_call(x, W1)
    degp = _deg_kernel(dst32)
    dis_col = _dis_call(degp).reshape(NP, 1)
    g1 = _scale_call(hw1, dis_col)
    p1 = _edge1(srcb, dstb, g1.reshape(NC * NP, 128), z128)
    g2 = _mid1_call(p1, g1, dis_col, b1r, W2)
    p2 = _edge64(src32, dst32, g2, z64)
    g3 = _mid2_call(p2, g2, dis_col, b2r, W3p)
    p3 = _edge64(src32, dst32, g3, z64)
    return _final_call(p3, g3, dis_col, b3r)
